# baseline (speedup 1.0000x reference)
import functools

import jax
import jax.numpy as jnp
from jax import lax
from jax.experimental import pallas as pl
from jax.experimental.pallas import tpu as pltpu
from jax.experimental.pallas import tpu_sc as plsc

EMB = 64
BATCH = 16384
MARGIN = 0.01

NC = 2
NS = 16
NW = NC * NS
PER_TILE = BATCH // NW
CHUNK = 128
NCHUNK = PER_TILE // CHUNK
NSTREAM = 5
ROW = EMB + 1


def _sqrt(x):
    i = lax.bitcast_convert_type(x, jnp.int32)
    y = lax.bitcast_convert_type(jnp.int32(0x5F3759DF) - (i >> 1), jnp.float32)
    for _ in range(3):
        y = y * (1.5 - 0.5 * x * y * y)
    return jnp.where(x > 0.0, x * y, 0.0)


def _sc_loss(table, idx, lvl):
    mesh = plsc.VectorSubcoreMesh(core_axis_name="c", subcore_axis_name="s")

    @functools.partial(
        pl.kernel,
        out_type=jax.ShapeDtypeStruct((BATCH,), jnp.float32),
        mesh=mesh,
        compiler_params=pltpu.CompilerParams(
            needs_layout_passes=False, use_tc_tiling_on_sc=False),
        scratch_types=[
            pltpu.VMEM((NSTREAM, NCHUNK, CHUNK), jnp.int32),
            pltpu.VMEM((PER_TILE,), jnp.int32),
            pltpu.VMEM((2, NSTREAM, CHUNK, ROW), jnp.float32),
            pltpu.VMEM((PER_TILE,), jnp.float32),
            pltpu.SemaphoreType.DMA,
            pltpu.SemaphoreType.DMA,
        ],
    )
    def k(table_hbm, idx_hbm, lvl_hbm, out_hbm, idx_v, lvl_v, bufs, out_v,
          sem0, sem1):
        sems = (sem0, sem1)
        wid = lax.axis_index("s") * NC + lax.axis_index("c")
        pltpu.sync_copy(idx_hbm.at[wid], idx_v)
        pltpu.sync_copy(lvl_hbm.at[wid], lvl_v)

        def fire(c, slot):
            return [
                pltpu.async_copy(
                    table_hbm.at[idx_v.at[s, c]],
                    bufs.at[slot, s], sems[slot])
                for s in range(NSTREAM)
            ]

        def compute_chunk(c, slot):
            bA = bufs.at[slot, 0]
            bB = bufs.at[slot, 1]
            bC = bufs.at[slot, 2]
            bD = bufs.at[slot, 3]
            bE = bufs.at[slot, 4]

            def group(g, _):
                rows = g * 16 + lax.iota(jnp.int32, 16)

                def dim_body(d, accs):
                    sa, sb, sab, sc_, sd, scd = accs
                    dv = jnp.full((16,), 0, jnp.int32) + d
                    a = plsc.load_gather(bA, [rows, dv])
                    b = plsc.load_gather(bB, [rows, dv])
                    cc = plsc.load_gather(bC, [rows, dv])
                    dd = plsc.load_gather(bD, [rows, dv])
                    eab = a - b
                    ecd = cc - dd
                    return (sa + a * a, sb + b * b, sab + eab * eab,
                            sc_ + cc * cc, sd + dd * dd, scd + ecd * ecd)

                z = jnp.zeros((16,), jnp.float32)
                sa, sb, sab, sc_, sd, scd = lax.fori_loop(
                    0, EMB, dim_body, (z, z, z, z, z, z))

                c64 = jnp.full((16,), EMB, jnp.int32)
                ra = jnp.abs(plsc.load_gather(bA, [rows, c64]))
                rb = jnp.abs(plsc.load_gather(bB, [rows, c64]))
                rc = jnp.abs(plsc.load_gather(bC, [rows, c64]))
                rd = jnp.abs(plsc.load_gather(bD, [rows, c64]))
                re = jnp.abs(plsc.load_gather(bE, [rows, c64]))

                euc = _sqrt(sab)
                na = _sqrt(sa)
                nb = _sqrt(sb)
                loss1 = (jnp.maximum(euc + ra - rb - MARGIN, 0.0)
                         + jnp.abs(na - 1.0) + jnp.abs(nb - 1.0))

                dst = _sqrt(scd)
                nc_ = _sqrt(sc_)
                nd = _sqrt(sd)
                lossd = (jnp.maximum(rc + rd - dst + MARGIN, 0.0)
                         + jnp.abs(nc_ - 1.0) + jnp.abs(nd - 1.0))

                off = c * CHUNK + g * 16
                lvl16 = lvl_v[pl.ds(off, 16)].astype(jnp.float32)
                lossr = jnp.maximum(_sqrt(lvl16) * 0.5 - re, 0.0)

                out_v[pl.ds(off, 16)] = loss1 + lossd + lossr
                return 0

            lax.fori_loop(0, CHUNK // 16, group, 0)

        pending = {0: fire(0, 0), 1: []}
        for c in range(NCHUNK):
            slot = c % 2
            for cp in pending[slot]:
                cp.wait()
            if c + 1 < NCHUNK:
                pending[1 - slot] = fire(c + 1, 1 - slot)
            compute_chunk(c, slot)

        pltpu.sync_copy(out_v, out_hbm.at[pl.ds(wid * PER_TILE, PER_TILE)])

    return k(table, idx, lvl)


def kernel(nf1, nf1_neg, dis, cl, cls_table):
    del nf1_neg
    idx = jnp.stack(
        [nf1[:, 0], nf1[:, 1], dis[:, 0], dis[:, 1], cl[:, 0]], axis=0)
    idx = idx.reshape(NSTREAM, NW, NCHUNK, CHUNK).transpose(1, 0, 2, 3)
    lvl = cl[:, 1].reshape(NW, PER_TILE)
    out = _sc_loss(cls_table, idx, lvl)
    return out.reshape(BATCH, 1)


# speedup vs baseline: 2.3784x; 2.3784x over previous
import functools

import jax
import jax.numpy as jnp
from jax import lax
from jax.experimental import pallas as pl
from jax.experimental.pallas import tpu as pltpu
from jax.experimental.pallas import tpu_sc as plsc

EMB = 64
BATCH = 16384
MARGIN = 0.01

NC = 2
NS = 16
NW = NC * NS
PER_TILE = BATCH // NW
CHUNK = 64
NCHUNK = PER_TILE // CHUNK
NSTREAM = 5
ROW = 128


def _sqrt(x):
    i = lax.bitcast_convert_type(x, jnp.int32)
    y = lax.bitcast_convert_type(jnp.int32(0x5F3759DF) - (i >> 1), jnp.float32)
    for _ in range(3):
        y = y * (1.5 - 0.5 * x * y * y)
    return jnp.where(x > 0.0, x * y, 0.0)


REPACK_BLK = 1024


def _repack(table_t):
    n = table_t.shape[1]
    grid = (pl.cdiv(n, REPACK_BLK),)

    def body(x_ref, o_ref):
        x = x_ref[...]
        pad = jnp.zeros((ROW - EMB - 1, REPACK_BLK), jnp.float32)
        o_ref[...] = jnp.transpose(jnp.concatenate([x, pad], axis=0), (1, 0))

    return pl.pallas_call(
        body,
        grid=grid,
        in_specs=[pl.BlockSpec((EMB + 1, REPACK_BLK), lambda i: (0, i))],
        out_specs=pl.BlockSpec((REPACK_BLK, ROW), lambda i: (i, 0)),
        out_shape=jax.ShapeDtypeStruct((n, ROW), jnp.float32),
    )(table_t)


def _sc_loss(table, idx, lvl):
    mesh = plsc.VectorSubcoreMesh(core_axis_name="c", subcore_axis_name="s")

    @functools.partial(
        pl.kernel,
        out_type=jax.ShapeDtypeStruct((BATCH,), jnp.float32),
        mesh=mesh,
        compiler_params=pltpu.CompilerParams(
            needs_layout_passes=False, use_tc_tiling_on_sc=True),
        scratch_types=[
            pltpu.VMEM((NSTREAM, NCHUNK, CHUNK), jnp.int32),
            pltpu.VMEM((PER_TILE,), jnp.int32),
            pltpu.VMEM((2, NSTREAM, CHUNK, ROW), jnp.float32),
            pltpu.VMEM((PER_TILE,), jnp.float32),
            pltpu.SemaphoreType.DMA,
            pltpu.SemaphoreType.DMA,
        ],
    )
    def k(table_hbm, idx_hbm, lvl_hbm, out_hbm, idx_v, lvl_v, bufs, out_v,
          sem0, sem1):
        sems = (sem0, sem1)
        wid = lax.axis_index("s") * NC + lax.axis_index("c")
        pltpu.sync_copy(idx_hbm.at[wid], idx_v)
        pltpu.sync_copy(lvl_hbm.at[wid], lvl_v)

        def fire(c, slot):
            return [
                pltpu.async_copy(
                    table_hbm.at[idx_v.at[s, c]],
                    bufs.at[slot, s], sems[slot])
                for s in range(NSTREAM)
            ]

        def compute_chunk(c, slot):
            bA = bufs.at[slot, 0]
            bB = bufs.at[slot, 1]
            bC = bufs.at[slot, 2]
            bD = bufs.at[slot, 3]
            bE = bufs.at[slot, 4]

            def group(g, _):
                rows = g * 16 + lax.iota(jnp.int32, 16)

                def dim_body(d, accs):
                    sa, sb, sab, sc_, sd, scd = accs
                    dv = jnp.full((16,), 0, jnp.int32) + d
                    a = plsc.load_gather(bA, [rows, dv])
                    b = plsc.load_gather(bB, [rows, dv])
                    cc = plsc.load_gather(bC, [rows, dv])
                    dd = plsc.load_gather(bD, [rows, dv])
                    eab = a - b
                    ecd = cc - dd
                    return (sa + a * a, sb + b * b, sab + eab * eab,
                            sc_ + cc * cc, sd + dd * dd, scd + ecd * ecd)

                z = jnp.zeros((16,), jnp.float32)
                sa, sb, sab, sc_, sd, scd = lax.fori_loop(
                    0, EMB, dim_body, (z, z, z, z, z, z))

                c64 = jnp.full((16,), EMB, jnp.int32)
                ra = jnp.abs(plsc.load_gather(bA, [rows, c64]))
                rb = jnp.abs(plsc.load_gather(bB, [rows, c64]))
                rc = jnp.abs(plsc.load_gather(bC, [rows, c64]))
                rd = jnp.abs(plsc.load_gather(bD, [rows, c64]))
                re = jnp.abs(plsc.load_gather(bE, [rows, c64]))

                euc = _sqrt(sab)
                na = _sqrt(sa)
                nb = _sqrt(sb)
                loss1 = (jnp.maximum(euc + ra - rb - MARGIN, 0.0)
                         + jnp.abs(na - 1.0) + jnp.abs(nb - 1.0))

                dst = _sqrt(scd)
                nc_ = _sqrt(sc_)
                nd = _sqrt(sd)
                lossd = (jnp.maximum(rc + rd - dst + MARGIN, 0.0)
                         + jnp.abs(nc_ - 1.0) + jnp.abs(nd - 1.0))

                off = c * CHUNK + g * 16
                lvl16 = lvl_v[pl.ds(off, 16)].astype(jnp.float32)
                lossr = jnp.maximum(_sqrt(lvl16) * 0.5 - re, 0.0)

                out_v[pl.ds(off, 16)] = loss1 + lossd + lossr
                return 0

            lax.fori_loop(0, CHUNK // 16, group, 0)

        pending = {0: fire(0, 0), 1: []}
        for c in range(NCHUNK):
            slot = c % 2
            for cp in pending[slot]:
                cp.wait()
            if c + 1 < NCHUNK:
                pending[1 - slot] = fire(c + 1, 1 - slot)
            compute_chunk(c, slot)

        pltpu.sync_copy(out_v, out_hbm.at[pl.ds(wid * PER_TILE, PER_TILE)])

    return k(table, idx, lvl)


def kernel(nf1, nf1_neg, dis, cl, cls_table):
    del nf1_neg
    idx = jnp.stack(
        [nf1[:, 0], nf1[:, 1], dis[:, 0], dis[:, 1], cl[:, 0]], axis=0)
    idx = idx.reshape(NSTREAM, NW, NCHUNK, CHUNK).transpose(1, 0, 2, 3)
    lvl = cl[:, 1].reshape(NW, PER_TILE)
    table128 = _repack(cls_table.T)
    out = _sc_loss(table128, idx, lvl)
    return out.reshape(BATCH, 1)


# speedup vs baseline: 4.4320x; 1.8634x over previous
import functools

import jax
import jax.numpy as jnp
from jax import lax
from jax.experimental import pallas as pl
from jax.experimental.pallas import tpu as pltpu
from jax.experimental.pallas import tpu_sc as plsc

EMB = 64
BATCH = 16384
MARGIN = 0.01

NC = 2
NS = 16
NW = NC * NS
PER_TILE = BATCH // NW
CHUNK = 64
NCHUNK = PER_TILE // CHUNK
NSTREAM = 5
ROW = 128


def _sqrt(x):
    i = lax.bitcast_convert_type(x, jnp.int32)
    y = lax.bitcast_convert_type(jnp.int32(0x5F3759DF) - (i >> 1), jnp.float32)
    for _ in range(3):
        y = y * (1.5 - 0.5 * x * y * y)
    return jnp.where(x > 0.0, x * y, 0.0)


REPACK_BLK = 4096


def _repack(table_t):
    n = table_t.shape[1]
    grid = (pl.cdiv(n, REPACK_BLK),)

    def body(x_ref, o_ref):
        x = x_ref[...]
        pad = jnp.zeros((ROW - EMB - 1, REPACK_BLK), jnp.float32)
        o_ref[...] = jnp.transpose(jnp.concatenate([x, pad], axis=0), (1, 0))

    return pl.pallas_call(
        body,
        grid=grid,
        in_specs=[pl.BlockSpec((EMB + 1, REPACK_BLK), lambda i: (0, i))],
        out_specs=pl.BlockSpec((REPACK_BLK, ROW), lambda i: (i, 0)),
        out_shape=jax.ShapeDtypeStruct((n, ROW), jnp.float32),
    )(table_t)


def _sc_loss(table, idx, lvl):
    mesh = plsc.VectorSubcoreMesh(core_axis_name="c", subcore_axis_name="s")

    @functools.partial(
        pl.kernel,
        out_type=jax.ShapeDtypeStruct((BATCH,), jnp.float32),
        mesh=mesh,
        compiler_params=pltpu.CompilerParams(
            needs_layout_passes=False, use_tc_tiling_on_sc=True),
        scratch_types=[
            pltpu.VMEM((NSTREAM, NCHUNK, CHUNK), jnp.int32),
            pltpu.VMEM((PER_TILE,), jnp.int32),
            pltpu.VMEM((2, NSTREAM, CHUNK, ROW), jnp.float32),
            pltpu.VMEM((PER_TILE,), jnp.float32),
            pltpu.SemaphoreType.DMA,
            pltpu.SemaphoreType.DMA,
        ],
    )
    def k(table_hbm, idx_hbm, lvl_hbm, out_hbm, idx_v, lvl_v, bufs, out_v,
          sem0, sem1):
        sems = (sem0, sem1)
        wid = lax.axis_index("s") * NC + lax.axis_index("c")
        pltpu.sync_copy(idx_hbm.at[wid], idx_v)
        pltpu.sync_copy(lvl_hbm.at[wid], lvl_v)

        def fire(c, slot):
            return [
                pltpu.async_copy(
                    table_hbm.at[idx_v.at[s, c]],
                    bufs.at[slot, s], sems[slot])
                for s in range(NSTREAM)
            ]

        def compute_chunk(c, slot):
            bA = bufs.at[slot, 0]
            bB = bufs.at[slot, 1]
            bC = bufs.at[slot, 2]
            bD = bufs.at[slot, 3]
            bE = bufs.at[slot, 4]

            def group(g, _):
                rows = g * 16 + lax.iota(jnp.int32, 16)

                def dim_body(i, accs):
                    sa, sb, sab, sc_, sd, scd = accs
                    for k in range(4):
                        dv = jnp.full((16,), 0, jnp.int32) + (i * 4 + k)
                        a = plsc.load_gather(bA, [rows, dv])
                        b = plsc.load_gather(bB, [rows, dv])
                        cc = plsc.load_gather(bC, [rows, dv])
                        dd = plsc.load_gather(bD, [rows, dv])
                        eab = a - b
                        ecd = cc - dd
                        sa = sa + a * a
                        sb = sb + b * b
                        sab = sab + eab * eab
                        sc_ = sc_ + cc * cc
                        sd = sd + dd * dd
                        scd = scd + ecd * ecd
                    return (sa, sb, sab, sc_, sd, scd)

                z = jnp.zeros((16,), jnp.float32)
                sa, sb, sab, sc_, sd, scd = lax.fori_loop(
                    0, EMB // 4, dim_body, (z, z, z, z, z, z))

                c64 = jnp.full((16,), EMB, jnp.int32)
                ra = jnp.abs(plsc.load_gather(bA, [rows, c64]))
                rb = jnp.abs(plsc.load_gather(bB, [rows, c64]))
                rc = jnp.abs(plsc.load_gather(bC, [rows, c64]))
                rd = jnp.abs(plsc.load_gather(bD, [rows, c64]))
                re = jnp.abs(plsc.load_gather(bE, [rows, c64]))

                euc = _sqrt(sab)
                na = _sqrt(sa)
                nb = _sqrt(sb)
                loss1 = (jnp.maximum(euc + ra - rb - MARGIN, 0.0)
                         + jnp.abs(na - 1.0) + jnp.abs(nb - 1.0))

                dst = _sqrt(scd)
                nc_ = _sqrt(sc_)
                nd = _sqrt(sd)
                lossd = (jnp.maximum(rc + rd - dst + MARGIN, 0.0)
                         + jnp.abs(nc_ - 1.0) + jnp.abs(nd - 1.0))

                off = c * CHUNK + g * 16
                lvl16 = lvl_v[pl.ds(off, 16)].astype(jnp.float32)
                lossr = jnp.maximum(_sqrt(lvl16) * 0.5 - re, 0.0)

                out_v[pl.ds(off, 16)] = loss1 + lossd + lossr
                return 0

            lax.fori_loop(0, CHUNK // 16, group, 0)

        pending = {0: fire(0, 0), 1: []}
        for c in range(NCHUNK):
            slot = c % 2
            for cp in pending[slot]:
                cp.wait()
            if c + 1 < NCHUNK:
                pending[1 - slot] = fire(c + 1, 1 - slot)
            compute_chunk(c, slot)

        pltpu.sync_copy(out_v, out_hbm.at[pl.ds(wid * PER_TILE, PER_TILE)])

    return k(table, idx, lvl)


def kernel(nf1, nf1_neg, dis, cl, cls_table):
    del nf1_neg
    idx = jnp.stack(
        [nf1[:, 0], nf1[:, 1], dis[:, 0], dis[:, 1], cl[:, 0]], axis=0)
    idx = idx.reshape(NSTREAM, NW, NCHUNK, CHUNK).transpose(1, 0, 2, 3)
    lvl = cl[:, 1].reshape(NW, PER_TILE)
    table128 = _repack(cls_table.T)
    out = _sc_loss(table128, idx, lvl)
    return out.reshape(BATCH, 1)


# speedup vs baseline: 5.3425x; 1.2054x over previous
import functools

import jax
import jax.numpy as jnp
from jax import lax
from jax.experimental import pallas as pl
from jax.experimental.pallas import tpu as pltpu
from jax.experimental.pallas import tpu_sc as plsc

EMB = 64
BATCH = 16384
MARGIN = 0.01

NC = 2
NS = 16
NW = NC * NS
PER_TILE = BATCH // NW
CHUNK = 64
NCHUNK = PER_TILE // CHUNK
NSTREAM = 5
ROW = 128


def _sqrt(x):
    i = lax.bitcast_convert_type(x, jnp.int32)
    y = lax.bitcast_convert_type(jnp.int32(0x5F3759DF) - (i >> 1), jnp.float32)
    for _ in range(3):
        y = y * (1.5 - 0.5 * x * y * y)
    return jnp.where(x > 0.0, x * y, 0.0)


REPACK_BLK = 8192


def _repack(table_t):
    n = table_t.shape[1]
    grid = (pl.cdiv(n, REPACK_BLK),)

    def body(x_ref, o_ref):
        x = x_ref[...]
        pad = jnp.zeros((ROW - EMB - 1, REPACK_BLK), jnp.float32)
        o_ref[...] = jnp.transpose(jnp.concatenate([x, pad], axis=0), (1, 0))

    return pl.pallas_call(
        body,
        grid=grid,
        in_specs=[pl.BlockSpec((EMB + 1, REPACK_BLK), lambda i: (0, i))],
        out_specs=pl.BlockSpec((REPACK_BLK, ROW), lambda i: (i, 0)),
        out_shape=jax.ShapeDtypeStruct((n, ROW), jnp.float32),
    )(table_t)


def _sc_loss(table, idx2, idxe, lvl):
    mesh = plsc.VectorSubcoreMesh(core_axis_name="c", subcore_axis_name="s")

    @functools.partial(
        pl.kernel,
        out_type=jax.ShapeDtypeStruct((BATCH,), jnp.float32),
        mesh=mesh,
        compiler_params=pltpu.CompilerParams(
            needs_layout_passes=False, use_tc_tiling_on_sc=True),
        scratch_types=[
            pltpu.VMEM((NCHUNK, 2, 2 * CHUNK), jnp.int32),
            pltpu.VMEM((NCHUNK, CHUNK), jnp.int32),
            pltpu.VMEM((PER_TILE,), jnp.int32),
            pltpu.VMEM((2, 2, 2 * CHUNK, ROW), jnp.float32),
            pltpu.VMEM((2, CHUNK, ROW), jnp.float32),
            pltpu.VMEM((PER_TILE,), jnp.float32),
            pltpu.SemaphoreType.DMA,
            pltpu.SemaphoreType.DMA,
        ],
    )
    def k(table_hbm, idx2_hbm, idxe_hbm, lvl_hbm, out_hbm,
          idx2_v, idxe_v, lvl_v, bufs, bufe, out_v, sem0, sem1):
        sems = (sem0, sem1)
        wid = lax.axis_index("s") * NC + lax.axis_index("c")
        pltpu.sync_copy(idx2_hbm.at[wid], idx2_v)
        pltpu.sync_copy(idxe_hbm.at[wid], idxe_v)
        pltpu.sync_copy(lvl_hbm.at[wid], lvl_v)

        def fire(c, slot):
            return [
                pltpu.async_copy(
                    table_hbm.at[idx2_v.at[c, 0]],
                    bufs.at[slot, 0], sems[slot]),
                pltpu.async_copy(
                    table_hbm.at[idx2_v.at[c, 1]],
                    bufs.at[slot, 1], sems[slot]),
                pltpu.async_copy(
                    table_hbm.at[idxe_v.at[c]],
                    bufe.at[slot], sems[slot]),
            ]

        def compute_chunk(c, slot):
            bAB = bufs.at[slot, 0]
            bCD = bufs.at[slot, 1]
            bE = bufe.at[slot]

            def group(g, _):
                rows = g * 16 + lax.iota(jnp.int32, 16)
                rows2 = rows + CHUNK

                def dim_body(i, accs):
                    sa, sb, sab, sc_, sd, scd = accs
                    for k in range(4):
                        dv = jnp.full((16,), 0, jnp.int32) + (i * 4 + k)
                        a = plsc.load_gather(bAB, [rows, dv])
                        b = plsc.load_gather(bAB, [rows2, dv])
                        cc = plsc.load_gather(bCD, [rows, dv])
                        dd = plsc.load_gather(bCD, [rows2, dv])
                        eab = a - b
                        ecd = cc - dd
                        sa = sa + a * a
                        sb = sb + b * b
                        sab = sab + eab * eab
                        sc_ = sc_ + cc * cc
                        sd = sd + dd * dd
                        scd = scd + ecd * ecd
                    return (sa, sb, sab, sc_, sd, scd)

                z = jnp.zeros((16,), jnp.float32)
                sa, sb, sab, sc_, sd, scd = lax.fori_loop(
                    0, EMB // 4, dim_body, (z, z, z, z, z, z))

                c64 = jnp.full((16,), EMB, jnp.int32)
                ra = jnp.abs(plsc.load_gather(bAB, [rows, c64]))
                rb = jnp.abs(plsc.load_gather(bAB, [rows2, c64]))
                rc = jnp.abs(plsc.load_gather(bCD, [rows, c64]))
                rd = jnp.abs(plsc.load_gather(bCD, [rows2, c64]))
                re = jnp.abs(plsc.load_gather(bE, [rows, c64]))

                euc = _sqrt(sab)
                na = _sqrt(sa)
                nb = _sqrt(sb)
                loss1 = (jnp.maximum(euc + ra - rb - MARGIN, 0.0)
                         + jnp.abs(na - 1.0) + jnp.abs(nb - 1.0))

                dst = _sqrt(scd)
                nc_ = _sqrt(sc_)
                nd = _sqrt(sd)
                lossd = (jnp.maximum(rc + rd - dst + MARGIN, 0.0)
                         + jnp.abs(nc_ - 1.0) + jnp.abs(nd - 1.0))

                off = c * CHUNK + g * 16
                lvl16 = lvl_v[pl.ds(off, 16)].astype(jnp.float32)
                lossr = jnp.maximum(_sqrt(lvl16) * 0.5 - re, 0.0)

                out_v[pl.ds(off, 16)] = loss1 + lossd + lossr
                return 0

            lax.fori_loop(0, CHUNK // 16, group, 0)

        pending = {0: fire(0, 0), 1: []}
        for c in range(NCHUNK):
            slot = c % 2
            for cp in pending[slot]:
                cp.wait()
            if c + 1 < NCHUNK:
                pending[1 - slot] = fire(c + 1, 1 - slot)
            compute_chunk(c, slot)

        pltpu.sync_copy(out_v, out_hbm.at[pl.ds(wid * PER_TILE, PER_TILE)])

    return k(table, idx2, idxe, lvl)


def kernel(nf1, nf1_neg, dis, cl, cls_table):
    del nf1_neg
    ab = jnp.concatenate(
        [nf1[:, 0].reshape(NW, NCHUNK, CHUNK),
         nf1[:, 1].reshape(NW, NCHUNK, CHUNK)], axis=-1)
    cd = jnp.concatenate(
        [dis[:, 0].reshape(NW, NCHUNK, CHUNK),
         dis[:, 1].reshape(NW, NCHUNK, CHUNK)], axis=-1)
    idx2 = jnp.stack([ab, cd], axis=2)
    idxe = cl[:, 0].reshape(NW, NCHUNK, CHUNK)
    lvl = cl[:, 1].reshape(NW, PER_TILE)
    table128 = _repack(cls_table.T)
    out = _sc_loss(table128, idx2, idxe, lvl)
    return out.reshape(BATCH, 1)


# speedup vs baseline: 5.5648x; 1.0416x over previous
import functools

import jax
import jax.numpy as jnp
from jax import lax
from jax.experimental import pallas as pl
from jax.experimental.pallas import tpu as pltpu
from jax.experimental.pallas import tpu_sc as plsc

EMB = 64
BATCH = 16384
MARGIN = 0.01

NC = 2
NS = 16
NW = NC * NS
PER_TILE = BATCH // NW
CHUNK = 64
NCHUNK = PER_TILE // CHUNK
NSTREAM = 5
ROW = 128


def _sqrt(x):
    i = lax.bitcast_convert_type(x, jnp.int32)
    y = lax.bitcast_convert_type(jnp.int32(0x5F3759DF) - (i >> 1), jnp.float32)
    for _ in range(3):
        y = y * (1.5 - 0.5 * x * y * y)
    return jnp.where(x > 0.0, x * y, 0.0)


REPACK_BLK = 16384
NBUF = 3


def _repack(table_t):
    n = table_t.shape[1]
    grid = (pl.cdiv(n, REPACK_BLK),)

    def body(x_ref, o_ref):
        x = x_ref[...]
        pad = jnp.zeros((ROW - EMB - 1, REPACK_BLK), jnp.float32)
        o_ref[...] = jnp.transpose(jnp.concatenate([x, pad], axis=0), (1, 0))

    return pl.pallas_call(
        body,
        grid=grid,
        in_specs=[pl.BlockSpec((EMB + 1, REPACK_BLK), lambda i: (0, i))],
        out_specs=pl.BlockSpec((REPACK_BLK, ROW), lambda i: (i, 0)),
        out_shape=jax.ShapeDtypeStruct((n, ROW), jnp.float32),
    )(table_t)


def _sc_loss(table, idx2, idxe, lvl):
    mesh = plsc.VectorSubcoreMesh(core_axis_name="c", subcore_axis_name="s")

    @functools.partial(
        pl.kernel,
        out_type=jax.ShapeDtypeStruct((BATCH,), jnp.float32),
        mesh=mesh,
        compiler_params=pltpu.CompilerParams(
            needs_layout_passes=False, use_tc_tiling_on_sc=True),
        scratch_types=[
            pltpu.VMEM((NCHUNK, 2, 2 * CHUNK), jnp.int32),
            pltpu.VMEM((NCHUNK, CHUNK), jnp.int32),
            pltpu.VMEM((PER_TILE,), jnp.int32),
            pltpu.VMEM((NBUF, 2, 2 * CHUNK, ROW), jnp.float32),
            pltpu.VMEM((NBUF, CHUNK, ROW), jnp.float32),
            pltpu.VMEM((PER_TILE,), jnp.float32),
            pltpu.SemaphoreType.DMA,
            pltpu.SemaphoreType.DMA,
            pltpu.SemaphoreType.DMA,
        ],
    )
    def k(table_hbm, idx2_hbm, idxe_hbm, lvl_hbm, out_hbm,
          idx2_v, idxe_v, lvl_v, bufs, bufe, out_v, sem0, sem1, sem2):
        sems = (sem0, sem1, sem2)
        wid = lax.axis_index("s") * NC + lax.axis_index("c")
        pltpu.sync_copy(idx2_hbm.at[wid], idx2_v)
        pltpu.sync_copy(idxe_hbm.at[wid], idxe_v)
        pltpu.sync_copy(lvl_hbm.at[wid], lvl_v)

        def fire(c, slot):
            return [
                pltpu.async_copy(
                    table_hbm.at[idx2_v.at[c, 0]],
                    bufs.at[slot, 0], sems[slot]),
                pltpu.async_copy(
                    table_hbm.at[idx2_v.at[c, 1]],
                    bufs.at[slot, 1], sems[slot]),
                pltpu.async_copy(
                    table_hbm.at[idxe_v.at[c]],
                    bufe.at[slot], sems[slot]),
            ]

        def compute_chunk(c, slot):
            bAB = bufs.at[slot, 0]
            bCD = bufs.at[slot, 1]
            bE = bufe.at[slot]

            def group(g, _):
                rows = g * 16 + lax.iota(jnp.int32, 16)
                rows2 = rows + CHUNK

                def dim_body(i, accs):
                    sa, sb, sab, sc_, sd, scd = accs
                    for k in range(4):
                        dv = jnp.full((16,), 0, jnp.int32) + (i * 4 + k)
                        a = plsc.load_gather(bAB, [rows, dv])
                        b = plsc.load_gather(bAB, [rows2, dv])
                        cc = plsc.load_gather(bCD, [rows, dv])
                        dd = plsc.load_gather(bCD, [rows2, dv])
                        eab = a - b
                        ecd = cc - dd
                        sa = sa + a * a
                        sb = sb + b * b
                        sab = sab + eab * eab
                        sc_ = sc_ + cc * cc
                        sd = sd + dd * dd
                        scd = scd + ecd * ecd
                    return (sa, sb, sab, sc_, sd, scd)

                z = jnp.zeros((16,), jnp.float32)
                sa, sb, sab, sc_, sd, scd = lax.fori_loop(
                    0, EMB // 4, dim_body, (z, z, z, z, z, z))

                c64 = jnp.full((16,), EMB, jnp.int32)
                ra = jnp.abs(plsc.load_gather(bAB, [rows, c64]))
                rb = jnp.abs(plsc.load_gather(bAB, [rows2, c64]))
                rc = jnp.abs(plsc.load_gather(bCD, [rows, c64]))
                rd = jnp.abs(plsc.load_gather(bCD, [rows2, c64]))
                re = jnp.abs(plsc.load_gather(bE, [rows, c64]))

                euc = _sqrt(sab)
                na = _sqrt(sa)
                nb = _sqrt(sb)
                loss1 = (jnp.maximum(euc + ra - rb - MARGIN, 0.0)
                         + jnp.abs(na - 1.0) + jnp.abs(nb - 1.0))

                dst = _sqrt(scd)
                nc_ = _sqrt(sc_)
                nd = _sqrt(sd)
                lossd = (jnp.maximum(rc + rd - dst + MARGIN, 0.0)
                         + jnp.abs(nc_ - 1.0) + jnp.abs(nd - 1.0))

                off = c * CHUNK + g * 16
                lvl16 = lvl_v[pl.ds(off, 16)].astype(jnp.float32)
                lossr = jnp.maximum(_sqrt(lvl16) * 0.5 - re, 0.0)

                out_v[pl.ds(off, 16)] = loss1 + lossd + lossr
                return 0

            lax.fori_loop(0, CHUNK // 16, group, 0)

        pending = {s: [] for s in range(NBUF)}
        for s in range(NBUF - 1):
            pending[s] = fire(s, s)
        for c in range(NCHUNK):
            slot = c % NBUF
            for cp in pending[slot]:
                cp.wait()
            nxt = c + NBUF - 1
            if nxt < NCHUNK:
                pending[nxt % NBUF] = fire(nxt, nxt % NBUF)
            compute_chunk(c, slot)

        pltpu.sync_copy(out_v, out_hbm.at[pl.ds(wid * PER_TILE, PER_TILE)])

    return k(table, idx2, idxe, lvl)


def kernel(nf1, nf1_neg, dis, cl, cls_table):
    del nf1_neg
    ab = jnp.concatenate(
        [nf1[:, 0].reshape(NW, NCHUNK, CHUNK),
         nf1[:, 1].reshape(NW, NCHUNK, CHUNK)], axis=-1)
    cd = jnp.concatenate(
        [dis[:, 0].reshape(NW, NCHUNK, CHUNK),
         dis[:, 1].reshape(NW, NCHUNK, CHUNK)], axis=-1)
    idx2 = jnp.stack([ab, cd], axis=2)
    idxe = cl[:, 0].reshape(NW, NCHUNK, CHUNK)
    lvl = cl[:, 1].reshape(NW, PER_TILE)
    table128 = _repack(cls_table.T)
    out = _sc_loss(table128, idx2, idxe, lvl)
    return out.reshape(BATCH, 1)


# speedup vs baseline: 6.8596x; 1.2327x over previous
import functools

import jax
import jax.numpy as jnp
from jax import lax
from jax.experimental import pallas as pl
from jax.experimental.pallas import tpu as pltpu
from jax.experimental.pallas import tpu_sc as plsc

EMB = 64
BATCH = 16384
MARGIN = 0.01

NC = 2
NS = 16
NW = NC * NS
PER_TILE = BATCH // NW
CHUNK = 64
NCHUNK = PER_TILE // CHUNK
NSTREAM = 5
ROW = 128


def _sqrt(x):
    i = lax.bitcast_convert_type(x, jnp.int32)
    y = lax.bitcast_convert_type(jnp.int32(0x5F3759DF) - (i >> 1), jnp.float32)
    for _ in range(3):
        y = y * (1.5 - 0.5 * x * y * y)
    return jnp.where(x > 0.0, x * y, 0.0)


REPACK_BLK = 16384
NBUF = 3


def _repack(table_t):
    n = table_t.shape[1]
    grid = (pl.cdiv(n, REPACK_BLK),)

    def body(x_ref, o_ref):
        x = x_ref[...]
        pad = jnp.zeros((ROW - EMB - 1, REPACK_BLK), jnp.float32)
        y = jnp.transpose(jnp.concatenate([x, pad], axis=0), (1, 0))
        o_ref[...] = y.astype(jnp.bfloat16)

    return pl.pallas_call(
        body,
        grid=grid,
        in_specs=[pl.BlockSpec((EMB + 1, REPACK_BLK), lambda i: (0, i))],
        out_specs=pl.BlockSpec((REPACK_BLK, ROW), lambda i: (i, 0)),
        out_shape=jax.ShapeDtypeStruct((n, ROW), jnp.bfloat16),
    )(table_t)


def _sc_loss(table, idx2, par2, idxe, pare, lvl):
    mesh = plsc.VectorSubcoreMesh(core_axis_name="c", subcore_axis_name="s")

    @functools.partial(
        pl.kernel,
        out_type=jax.ShapeDtypeStruct((BATCH,), jnp.float32),
        mesh=mesh,
        compiler_params=pltpu.CompilerParams(
            needs_layout_passes=False, use_tc_tiling_on_sc=True),
        scratch_types=[
            pltpu.VMEM((NCHUNK, 2, 2 * CHUNK), jnp.int32),
            pltpu.VMEM((NCHUNK, 2, 2 * CHUNK), jnp.int32),
            pltpu.VMEM((NCHUNK, CHUNK), jnp.int32),
            pltpu.VMEM((NCHUNK, CHUNK), jnp.int32),
            pltpu.VMEM((PER_TILE,), jnp.int32),
            pltpu.VMEM((NBUF, 2, 2 * CHUNK, 1, ROW), jnp.int32),
            pltpu.VMEM((NBUF, CHUNK, 1, ROW), jnp.int32),
            pltpu.VMEM((PER_TILE,), jnp.float32),
            pltpu.SemaphoreType.DMA,
            pltpu.SemaphoreType.DMA,
            pltpu.SemaphoreType.DMA,
        ],
    )
    def k(table_hbm, idx2_hbm, par2_hbm, idxe_hbm, pare_hbm, lvl_hbm, out_hbm,
          idx2_v, par2_v, idxe_v, pare_v, lvl_v, bufs, bufe, out_v,
          sem0, sem1, sem2):
        sems = (sem0, sem1, sem2)
        wid = lax.axis_index("s") * NC + lax.axis_index("c")
        pltpu.sync_copy(idx2_hbm.at[wid], idx2_v)
        pltpu.sync_copy(par2_hbm.at[wid], par2_v)
        pltpu.sync_copy(idxe_hbm.at[wid], idxe_v)
        pltpu.sync_copy(pare_hbm.at[wid], pare_v)
        pltpu.sync_copy(lvl_hbm.at[wid], lvl_v)

        def fire(c, slot):
            ti = table_hbm.bitcast(jnp.int32)
            return [
                pltpu.async_copy(
                    ti.at[idx2_v.at[c, 0]], bufs.at[slot, 0], sems[slot]),
                pltpu.async_copy(
                    ti.at[idx2_v.at[c, 1]], bufs.at[slot, 1], sems[slot]),
                pltpu.async_copy(
                    ti.at[idxe_v.at[c]], bufe.at[slot], sems[slot]),
            ]

        def compute_chunk(c, slot):
            bAB = bufs.at[slot, 0]
            bCD = bufs.at[slot, 1]
            bE = bufe.at[slot]
            himask = jnp.full((16,), -65536, jnp.int32)

            def group(g, _):
                rows = g * 16 + lax.iota(jnp.int32, 16)
                rows2 = rows + CHUNK
                zv = jnp.zeros((16,), jnp.int32)
                sh_a = (1 - par2_v[c, 0, pl.ds(g * 16, 16)]) * 16
                sh_b = (1 - par2_v[c, 0, pl.ds(CHUNK + g * 16, 16)]) * 16
                sh_c = (1 - par2_v[c, 1, pl.ds(g * 16, 16)]) * 16
                sh_d = (1 - par2_v[c, 1, pl.ds(CHUNK + g * 16, 16)]) * 16
                sh_e = (1 - pare_v[c, pl.ds(g * 16, 16)]) * 16

                def bf(w, sh):
                    return plsc.bitcast((w << sh) & himask, jnp.float32)

                def dim_body(i, accs):
                    sa, sb, sab, sc_, sd, scd = accs
                    for k in range(2):
                        dv = jnp.full((16,), 0, jnp.int32) + (i * 2 + k)
                        a = bf(plsc.load_gather(bAB, [rows, zv, dv]), sh_a)
                        b = bf(plsc.load_gather(bAB, [rows2, zv, dv]), sh_b)
                        cc = bf(plsc.load_gather(bCD, [rows, zv, dv]), sh_c)
                        dd = bf(plsc.load_gather(bCD, [rows2, zv, dv]), sh_d)
                        eab = a - b
                        ecd = cc - dd
                        sa = sa + a * a
                        sb = sb + b * b
                        sab = sab + eab * eab
                        sc_ = sc_ + cc * cc
                        sd = sd + dd * dd
                        scd = scd + ecd * ecd
                    return (sa, sb, sab, sc_, sd, scd)

                z = jnp.zeros((16,), jnp.float32)
                sa, sb, sab, sc_, sd, scd = lax.fori_loop(
                    0, EMB // 2, dim_body, (z, z, z, z, z, z))

                c64 = jnp.full((16,), EMB, jnp.int32)
                ra = jnp.abs(bf(plsc.load_gather(bAB, [rows, zv, c64]), sh_a))
                rb = jnp.abs(bf(plsc.load_gather(bAB, [rows2, zv, c64]), sh_b))
                rc = jnp.abs(bf(plsc.load_gather(bCD, [rows, zv, c64]), sh_c))
                rd = jnp.abs(bf(plsc.load_gather(bCD, [rows2, zv, c64]), sh_d))
                re = jnp.abs(bf(plsc.load_gather(bE, [rows, zv, c64]), sh_e))

                euc = _sqrt(sab)
                na = _sqrt(sa)
                nb = _sqrt(sb)
                loss1 = (jnp.maximum(euc + ra - rb - MARGIN, 0.0)
                         + jnp.abs(na - 1.0) + jnp.abs(nb - 1.0))

                dst = _sqrt(scd)
                nc_ = _sqrt(sc_)
                nd = _sqrt(sd)
                lossd = (jnp.maximum(rc + rd - dst + MARGIN, 0.0)
                         + jnp.abs(nc_ - 1.0) + jnp.abs(nd - 1.0))

                off = c * CHUNK + g * 16
                lvl16 = lvl_v[pl.ds(off, 16)].astype(jnp.float32)
                lossr = jnp.maximum(_sqrt(lvl16) * 0.5 - re, 0.0)

                out_v[pl.ds(off, 16)] = loss1 + lossd + lossr
                return 0

            lax.fori_loop(0, CHUNK // 16, group, 0)

        pending = {s: [] for s in range(NBUF)}
        for s in range(NBUF - 1):
            pending[s] = fire(s, s)
        for c in range(NCHUNK):
            slot = c % NBUF
            for cp in pending[slot]:
                cp.wait()
            nxt = c + NBUF - 1
            if nxt < NCHUNK:
                pending[nxt % NBUF] = fire(nxt, nxt % NBUF)
            compute_chunk(c, slot)

        pltpu.sync_copy(out_v, out_hbm.at[pl.ds(wid * PER_TILE, PER_TILE)])

    return k(table, idx2, par2, idxe, pare, lvl)


def kernel(nf1, nf1_neg, dis, cl, cls_table):
    del nf1_neg
    ab = jnp.concatenate(
        [nf1[:, 0].reshape(NW, NCHUNK, CHUNK),
         nf1[:, 1].reshape(NW, NCHUNK, CHUNK)], axis=-1)
    cd = jnp.concatenate(
        [dis[:, 0].reshape(NW, NCHUNK, CHUNK),
         dis[:, 1].reshape(NW, NCHUNK, CHUNK)], axis=-1)
    full = jnp.stack([ab, cd], axis=2)
    idx2 = full >> 1
    par2 = full & 1
    e = cl[:, 0].reshape(NW, NCHUNK, CHUNK)
    idxe = e >> 1
    pare = e & 1
    lvl = cl[:, 1].reshape(NW, PER_TILE)
    table_bf = _repack(cls_table.T)
    table3 = table_bf.reshape(1000000 // 2, 2, ROW)
    out = _sc_loss(table3, idx2, par2, idxe, pare, lvl)
    return out.reshape(BATCH, 1)


# speedup vs baseline: 8.8230x; 1.2862x over previous
import functools

import jax
import jax.numpy as jnp
from jax import lax
from jax.experimental import pallas as pl
from jax.experimental.pallas import tpu as pltpu
from jax.experimental.pallas import tpu_sc as plsc

EMB = 64
BATCH = 16384
MARGIN = 0.01

NC = 2
NS = 16
NW = NC * NS
PER_TILE = BATCH // NW
CHUNK = 64
NCHUNK = PER_TILE // CHUNK
NSTREAM = 5
ROW = 128


def _sqrt(x):
    i = lax.bitcast_convert_type(x, jnp.int32)
    y = lax.bitcast_convert_type(jnp.int32(0x5F3759DF) - (i >> 1), jnp.float32)
    for _ in range(3):
        y = y * (1.5 - 0.5 * x * y * y)
    return jnp.where(x > 0.0, x * y, 0.0)


REPACK_BLK = 16384
NBUF = 3


def _repack(table_t):
    n = table_t.shape[1]
    grid = (pl.cdiv(n, REPACK_BLK),)

    def body(x_ref, o_ref):
        x = x_ref[...]
        pad = jnp.zeros((ROW - EMB - 1, REPACK_BLK), jnp.float32)
        y = jnp.transpose(jnp.concatenate([x, pad], axis=0), (1, 0))
        o_ref[...] = y.astype(jnp.bfloat16)

    return pl.pallas_call(
        body,
        grid=grid,
        in_specs=[pl.BlockSpec((EMB + 1, REPACK_BLK), lambda i: (0, i))],
        out_specs=pl.BlockSpec((REPACK_BLK, ROW), lambda i: (i, 0)),
        out_shape=jax.ShapeDtypeStruct((n, ROW), jnp.bfloat16),
    )(table_t)


def _sc_loss(table, idx2, par2, idxe, pare, lvl):
    mesh = plsc.VectorSubcoreMesh(core_axis_name="c", subcore_axis_name="s")

    @functools.partial(
        pl.kernel,
        out_type=jax.ShapeDtypeStruct((BATCH,), jnp.float32),
        mesh=mesh,
        compiler_params=pltpu.CompilerParams(
            needs_layout_passes=False, use_tc_tiling_on_sc=True),
        scratch_types=[
            pltpu.VMEM((NCHUNK, 2, 2 * CHUNK), jnp.int32),
            pltpu.VMEM((NCHUNK, 2, 2 * CHUNK), jnp.int32),
            pltpu.VMEM((NCHUNK, CHUNK), jnp.int32),
            pltpu.VMEM((NCHUNK, CHUNK), jnp.int32),
            pltpu.VMEM((PER_TILE,), jnp.int32),
            pltpu.VMEM((NBUF, 2, 2 * CHUNK, 1, ROW), jnp.int32),
            pltpu.VMEM((NBUF, CHUNK, 1, ROW), jnp.int32),
            pltpu.VMEM((PER_TILE,), jnp.float32),
            pltpu.SemaphoreType.DMA,
            pltpu.SemaphoreType.DMA,
            pltpu.SemaphoreType.DMA,
        ],
    )
    def k(table_hbm, idx2_hbm, par2_hbm, idxe_hbm, pare_hbm, lvl_hbm, out_hbm,
          idx2_v, par2_v, idxe_v, pare_v, lvl_v, bufs, bufe, out_v,
          sem0, sem1, sem2):
        sems = (sem0, sem1, sem2)
        wid = lax.axis_index("s") * NC + lax.axis_index("c")
        pltpu.sync_copy(idx2_hbm.at[wid], idx2_v)
        pltpu.sync_copy(par2_hbm.at[wid], par2_v)
        pltpu.sync_copy(idxe_hbm.at[wid], idxe_v)
        pltpu.sync_copy(pare_hbm.at[wid], pare_v)
        pltpu.sync_copy(lvl_hbm.at[wid], lvl_v)

        def fire(c, slot):
            ti = table_hbm.bitcast(jnp.int32)
            return [
                pltpu.async_copy(
                    ti.at[idx2_v.at[c, 0]], bufs.at[slot, 0], sems[slot]),
                pltpu.async_copy(
                    ti.at[idx2_v.at[c, 1]], bufs.at[slot, 1], sems[slot]),
                pltpu.async_copy(
                    ti.at[idxe_v.at[c]], bufe.at[slot], sems[slot]),
            ]

        def compute_chunk(c, slot):
            bAB = bufs.at[slot, 0]
            bCD = bufs.at[slot, 1]
            bE = bufe.at[slot]
            himask = jnp.full((16,), -65536, jnp.int32)

            def group(g, _):
                rows = g * 16 + lax.iota(jnp.int32, 16)
                rows2 = rows + CHUNK
                zv = jnp.zeros((16,), jnp.int32)
                sh_a = (1 - par2_v[c, 0, pl.ds(g * 16, 16)]) * 16
                sh_b = (1 - par2_v[c, 0, pl.ds(CHUNK + g * 16, 16)]) * 16
                sh_c = (1 - par2_v[c, 1, pl.ds(g * 16, 16)]) * 16
                sh_d = (1 - par2_v[c, 1, pl.ds(CHUNK + g * 16, 16)]) * 16
                sh_e = (1 - pare_v[c, pl.ds(g * 16, 16)]) * 16

                def bf(w, sh):
                    return plsc.bitcast((w << sh) & himask, jnp.float32)

                lane = lax.iota(jnp.int32, 16)
                d63 = jnp.full((16,), 63, jnp.int32)

                def dim_body(i, accs):
                    sa, sb, sab, sc_, sd, scd = accs
                    for k in range(2):
                        dv = (lane + (i * 2 + k)) & d63
                        a = bf(plsc.load_gather(bAB, [rows, zv, dv]), sh_a)
                        b = bf(plsc.load_gather(bAB, [rows2, zv, dv]), sh_b)
                        cc = bf(plsc.load_gather(bCD, [rows, zv, dv]), sh_c)
                        dd = bf(plsc.load_gather(bCD, [rows2, zv, dv]), sh_d)
                        eab = a - b
                        ecd = cc - dd
                        sa = sa + a * a
                        sb = sb + b * b
                        sab = sab + eab * eab
                        sc_ = sc_ + cc * cc
                        sd = sd + dd * dd
                        scd = scd + ecd * ecd
                    return (sa, sb, sab, sc_, sd, scd)

                z = jnp.zeros((16,), jnp.float32)
                sa, sb, sab, sc_, sd, scd = lax.fori_loop(
                    0, EMB // 2, dim_body, (z, z, z, z, z, z))

                c64 = jnp.full((16,), EMB, jnp.int32)
                ra = jnp.abs(bf(plsc.load_gather(bAB, [rows, zv, c64]), sh_a))
                rb = jnp.abs(bf(plsc.load_gather(bAB, [rows2, zv, c64]), sh_b))
                rc = jnp.abs(bf(plsc.load_gather(bCD, [rows, zv, c64]), sh_c))
                rd = jnp.abs(bf(plsc.load_gather(bCD, [rows2, zv, c64]), sh_d))
                re = jnp.abs(bf(plsc.load_gather(bE, [rows, zv, c64]), sh_e))

                euc = _sqrt(sab)
                na = _sqrt(sa)
                nb = _sqrt(sb)
                loss1 = (jnp.maximum(euc + ra - rb - MARGIN, 0.0)
                         + jnp.abs(na - 1.0) + jnp.abs(nb - 1.0))

                dst = _sqrt(scd)
                nc_ = _sqrt(sc_)
                nd = _sqrt(sd)
                lossd = (jnp.maximum(rc + rd - dst + MARGIN, 0.0)
                         + jnp.abs(nc_ - 1.0) + jnp.abs(nd - 1.0))

                off = c * CHUNK + g * 16
                lvl16 = lvl_v[pl.ds(off, 16)].astype(jnp.float32)
                lossr = jnp.maximum(_sqrt(lvl16) * 0.5 - re, 0.0)

                out_v[pl.ds(off, 16)] = loss1 + lossd + lossr
                return 0

            lax.fori_loop(0, CHUNK // 16, group, 0)

        pending = {s: [] for s in range(NBUF)}
        for s in range(NBUF - 1):
            pending[s] = fire(s, s)
        for c in range(NCHUNK):
            slot = c % NBUF
            for cp in pending[slot]:
                cp.wait()
            nxt = c + NBUF - 1
            if nxt < NCHUNK:
                pending[nxt % NBUF] = fire(nxt, nxt % NBUF)
            compute_chunk(c, slot)

        pltpu.sync_copy(out_v, out_hbm.at[pl.ds(wid * PER_TILE, PER_TILE)])

    return k(table, idx2, par2, idxe, pare, lvl)


def kernel(nf1, nf1_neg, dis, cl, cls_table):
    del nf1_neg
    ab = jnp.concatenate(
        [nf1[:, 0].reshape(NW, NCHUNK, CHUNK),
         nf1[:, 1].reshape(NW, NCHUNK, CHUNK)], axis=-1)
    cd = jnp.concatenate(
        [dis[:, 0].reshape(NW, NCHUNK, CHUNK),
         dis[:, 1].reshape(NW, NCHUNK, CHUNK)], axis=-1)
    full = jnp.stack([ab, cd], axis=2)
    idx2 = full >> 1
    par2 = full & 1
    e = cl[:, 0].reshape(NW, NCHUNK, CHUNK)
    idxe = e >> 1
    pare = e & 1
    lvl = cl[:, 1].reshape(NW, PER_TILE)
    table_bf = _repack(cls_table.T)
    table3 = table_bf.reshape(1000000 // 2, 2, ROW)
    out = _sc_loss(table3, idx2, par2, idxe, pare, lvl)
    return out.reshape(BATCH, 1)


# speedup vs baseline: 9.0339x; 1.0239x over previous
import functools

import jax
import jax.numpy as jnp
from jax import lax
from jax.experimental import pallas as pl
from jax.experimental.pallas import tpu as pltpu
from jax.experimental.pallas import tpu_sc as plsc

EMB = 64
BATCH = 16384
MARGIN = 0.01

NC = 2
NS = 16
NW = NC * NS
PER_TILE = BATCH // NW
CHUNK = 64
NCHUNK = PER_TILE // CHUNK
NSTREAM = 5
ROW = 128


def _sqrt(x):
    i = lax.bitcast_convert_type(x, jnp.int32)
    y = lax.bitcast_convert_type(jnp.int32(0x5F3759DF) - (i >> 1), jnp.float32)
    for _ in range(3):
        y = y * (1.5 - 0.5 * x * y * y)
    return jnp.where(x > 0.0, x * y, 0.0)


REPACK_BLK = 32768
NBUF = 3


def _repack(table_t):
    n = table_t.shape[1]
    grid = (pl.cdiv(n, REPACK_BLK),)

    def body(x_ref, o_ref):
        x = x_ref[...]
        pad = jnp.zeros((ROW - EMB - 1, REPACK_BLK), jnp.float32)
        y = jnp.transpose(jnp.concatenate([x, pad], axis=0), (1, 0))
        o_ref[...] = y.astype(jnp.bfloat16)

    return pl.pallas_call(
        body,
        grid=grid,
        in_specs=[pl.BlockSpec((EMB + 1, REPACK_BLK), lambda i: (0, i))],
        out_specs=pl.BlockSpec((REPACK_BLK, ROW), lambda i: (i, 0)),
        out_shape=jax.ShapeDtypeStruct((n, ROW), jnp.bfloat16),
    )(table_t)


def _sc_loss(table, idx2, par2, idxe, pare, lvl):
    mesh = plsc.VectorSubcoreMesh(core_axis_name="c", subcore_axis_name="s")

    @functools.partial(
        pl.kernel,
        out_type=jax.ShapeDtypeStruct((BATCH,), jnp.float32),
        mesh=mesh,
        compiler_params=pltpu.CompilerParams(
            needs_layout_passes=False, use_tc_tiling_on_sc=True),
        scratch_types=[
            pltpu.VMEM((NCHUNK, 2, 2 * CHUNK), jnp.int32),
            pltpu.VMEM((NCHUNK, 2, 2 * CHUNK), jnp.int32),
            pltpu.VMEM((NCHUNK, CHUNK), jnp.int32),
            pltpu.VMEM((NCHUNK, CHUNK), jnp.int32),
            pltpu.VMEM((PER_TILE,), jnp.int32),
            pltpu.VMEM((NBUF, 2, 2 * CHUNK, 1, ROW), jnp.int32),
            pltpu.VMEM((NBUF, CHUNK, 1, ROW), jnp.int32),
            pltpu.VMEM((PER_TILE,), jnp.float32),
            pltpu.SemaphoreType.DMA,
            pltpu.SemaphoreType.DMA,
            pltpu.SemaphoreType.DMA,
        ],
    )
    def k(table_hbm, idx2_hbm, par2_hbm, idxe_hbm, pare_hbm, lvl_hbm, out_hbm,
          idx2_v, par2_v, idxe_v, pare_v, lvl_v, bufs, bufe, out_v,
          sem0, sem1, sem2):
        sems = (sem0, sem1, sem2)
        wid = lax.axis_index("s") * NC + lax.axis_index("c")
        pltpu.sync_copy(idx2_hbm.at[wid], idx2_v)
        pltpu.sync_copy(par2_hbm.at[wid], par2_v)
        pltpu.sync_copy(idxe_hbm.at[wid], idxe_v)
        pltpu.sync_copy(pare_hbm.at[wid], pare_v)
        pltpu.sync_copy(lvl_hbm.at[wid], lvl_v)

        def fire(c, slot):
            ti = table_hbm.bitcast(jnp.int32)
            return [
                pltpu.async_copy(
                    ti.at[idx2_v.at[c, 0]], bufs.at[slot, 0], sems[slot]),
                pltpu.async_copy(
                    ti.at[idx2_v.at[c, 1]], bufs.at[slot, 1], sems[slot]),
                pltpu.async_copy(
                    ti.at[idxe_v.at[c]], bufe.at[slot], sems[slot]),
            ]

        def compute_chunk(c, slot):
            bAB = bufs.at[slot, 0]
            bCD = bufs.at[slot, 1]
            bE = bufe.at[slot]
            himask = jnp.full((16,), -65536, jnp.int32)

            def group(g, _):
                rows = g * 16 + lax.iota(jnp.int32, 16)
                rows2 = rows + CHUNK
                zv = jnp.zeros((16,), jnp.int32)
                sh_a = (1 - par2_v[c, 0, pl.ds(g * 16, 16)]) * 16
                sh_b = (1 - par2_v[c, 0, pl.ds(CHUNK + g * 16, 16)]) * 16
                sh_c = (1 - par2_v[c, 1, pl.ds(g * 16, 16)]) * 16
                sh_d = (1 - par2_v[c, 1, pl.ds(CHUNK + g * 16, 16)]) * 16
                sh_e = (1 - pare_v[c, pl.ds(g * 16, 16)]) * 16

                def bf(w, sh):
                    return plsc.bitcast((w << sh) & himask, jnp.float32)

                lane = lax.iota(jnp.int32, 16)
                d63 = jnp.full((16,), 63, jnp.int32)

                def dim_body(i, accs):
                    sa, sb, sab, sc_, sd, scd = accs
                    for k in range(2):
                        dv = (lane + (i * 2 + k)) & d63
                        a = bf(plsc.load_gather(bAB, [rows, zv, dv]), sh_a)
                        b = bf(plsc.load_gather(bAB, [rows2, zv, dv]), sh_b)
                        cc = bf(plsc.load_gather(bCD, [rows, zv, dv]), sh_c)
                        dd = bf(plsc.load_gather(bCD, [rows2, zv, dv]), sh_d)
                        eab = a - b
                        ecd = cc - dd
                        sa = sa + a * a
                        sb = sb + b * b
                        sab = sab + eab * eab
                        sc_ = sc_ + cc * cc
                        sd = sd + dd * dd
                        scd = scd + ecd * ecd
                    return (sa, sb, sab, sc_, sd, scd)

                z = jnp.zeros((16,), jnp.float32)
                sa, sb, sab, sc_, sd, scd = lax.fori_loop(
                    0, EMB // 2, dim_body, (z, z, z, z, z, z))

                c64 = jnp.full((16,), EMB, jnp.int32)
                ra = jnp.abs(bf(plsc.load_gather(bAB, [rows, zv, c64]), sh_a))
                rb = jnp.abs(bf(plsc.load_gather(bAB, [rows2, zv, c64]), sh_b))
                rc = jnp.abs(bf(plsc.load_gather(bCD, [rows, zv, c64]), sh_c))
                rd = jnp.abs(bf(plsc.load_gather(bCD, [rows2, zv, c64]), sh_d))
                re = jnp.abs(bf(plsc.load_gather(bE, [rows, zv, c64]), sh_e))

                euc = _sqrt(sab)
                na = _sqrt(sa)
                nb = _sqrt(sb)
                loss1 = (jnp.maximum(euc + ra - rb - MARGIN, 0.0)
                         + jnp.abs(na - 1.0) + jnp.abs(nb - 1.0))

                dst = _sqrt(scd)
                nc_ = _sqrt(sc_)
                nd = _sqrt(sd)
                lossd = (jnp.maximum(rc + rd - dst + MARGIN, 0.0)
                         + jnp.abs(nc_ - 1.0) + jnp.abs(nd - 1.0))

                off = c * CHUNK + g * 16
                lvl16 = lvl_v[pl.ds(off, 16)].astype(jnp.float32)
                lossr = jnp.maximum(_sqrt(lvl16) * 0.5 - re, 0.0)

                out_v[pl.ds(off, 16)] = loss1 + lossd + lossr
                return 0

            lax.fori_loop(0, CHUNK // 16, group, 0)

        pending = {s: [] for s in range(NBUF)}
        for s in range(NBUF - 1):
            pending[s] = fire(s, s)
        for c in range(NCHUNK):
            slot = c % NBUF
            for cp in pending[slot]:
                cp.wait()
            nxt = c + NBUF - 1
            if nxt < NCHUNK:
                pending[nxt % NBUF] = fire(nxt, nxt % NBUF)
            compute_chunk(c, slot)

        pltpu.sync_copy(out_v, out_hbm.at[pl.ds(wid * PER_TILE, PER_TILE)])

    return k(table, idx2, par2, idxe, pare, lvl)


def kernel(nf1, nf1_neg, dis, cl, cls_table):
    del nf1_neg
    ab = jnp.concatenate(
        [nf1[:, 0].reshape(NW, NCHUNK, CHUNK),
         nf1[:, 1].reshape(NW, NCHUNK, CHUNK)], axis=-1)
    cd = jnp.concatenate(
        [dis[:, 0].reshape(NW, NCHUNK, CHUNK),
         dis[:, 1].reshape(NW, NCHUNK, CHUNK)], axis=-1)
    full = jnp.stack([ab, cd], axis=2)
    idx2 = full >> 1
    par2 = full & 1
    e = cl[:, 0].reshape(NW, NCHUNK, CHUNK)
    idxe = e >> 1
    pare = e & 1
    lvl = cl[:, 1].reshape(NW, PER_TILE)
    table_bf = _repack(cls_table.T)
    table3 = table_bf.reshape(1000000 // 2, 2, ROW)
    out = _sc_loss(table3, idx2, par2, idxe, pare, lvl)
    return out.reshape(BATCH, 1)


# speedup vs baseline: 9.0748x; 1.0045x over previous
import functools

import jax
import jax.numpy as jnp
from jax import lax
from jax.experimental import pallas as pl
from jax.experimental.pallas import tpu as pltpu
from jax.experimental.pallas import tpu_sc as plsc

EMB = 64
BATCH = 16384
MARGIN = 0.01

NC = 2
NS = 16
NW = NC * NS
PER_TILE = BATCH // NW
CHUNK = 64
NCHUNK = PER_TILE // CHUNK
NSTREAM = 5
ROW = 128


def _sqrt(x):
    i = lax.bitcast_convert_type(x, jnp.int32)
    y = lax.bitcast_convert_type(jnp.int32(0x5F3759DF) - (i >> 1), jnp.float32)
    for _ in range(3):
        y = y * (1.5 - 0.5 * x * y * y)
    return jnp.where(x > 0.0, x * y, 0.0)


REPACK_BLK = 49152
NBUF = 3


def _repack(table_t):
    n = table_t.shape[1]
    grid = (pl.cdiv(n, REPACK_BLK),)

    def body(x_ref, o_ref):
        x = x_ref[...]
        pad = jnp.zeros((ROW - EMB - 1, REPACK_BLK), jnp.float32)
        y = jnp.transpose(jnp.concatenate([x, pad], axis=0), (1, 0))
        o_ref[...] = y.astype(jnp.bfloat16)

    return pl.pallas_call(
        body,
        grid=grid,
        in_specs=[pl.BlockSpec((EMB + 1, REPACK_BLK), lambda i: (0, i))],
        out_specs=pl.BlockSpec((REPACK_BLK, ROW), lambda i: (i, 0)),
        out_shape=jax.ShapeDtypeStruct((n, ROW), jnp.bfloat16),
    )(table_t)


def _sc_loss(table, idx2, par2, idxe, pare, lvl):
    mesh = plsc.VectorSubcoreMesh(core_axis_name="c", subcore_axis_name="s")

    @functools.partial(
        pl.kernel,
        out_type=jax.ShapeDtypeStruct((BATCH,), jnp.float32),
        mesh=mesh,
        compiler_params=pltpu.CompilerParams(
            needs_layout_passes=False, use_tc_tiling_on_sc=True),
        scratch_types=[
            pltpu.VMEM((NCHUNK, 2, 2 * CHUNK), jnp.int32),
            pltpu.VMEM((NCHUNK, 2, 2 * CHUNK), jnp.int32),
            pltpu.VMEM((NCHUNK, CHUNK), jnp.int32),
            pltpu.VMEM((NCHUNK, CHUNK), jnp.int32),
            pltpu.VMEM((PER_TILE,), jnp.int32),
            pltpu.VMEM((NBUF, 2, 2 * CHUNK, 1, ROW), jnp.int32),
            pltpu.VMEM((NBUF, CHUNK, 1, ROW), jnp.int32),
            pltpu.VMEM((PER_TILE,), jnp.float32),
            pltpu.SemaphoreType.DMA,
            pltpu.SemaphoreType.DMA,
            pltpu.SemaphoreType.DMA,
        ],
    )
    def k(table_hbm, idx2_hbm, par2_hbm, idxe_hbm, pare_hbm, lvl_hbm, out_hbm,
          idx2_v, par2_v, idxe_v, pare_v, lvl_v, bufs, bufe, out_v,
          sem0, sem1, sem2):
        sems = (sem0, sem1, sem2)
        wid = lax.axis_index("s") * NC + lax.axis_index("c")
        pltpu.sync_copy(idx2_hbm.at[wid], idx2_v)
        pltpu.sync_copy(par2_hbm.at[wid], par2_v)
        pltpu.sync_copy(idxe_hbm.at[wid], idxe_v)
        pltpu.sync_copy(pare_hbm.at[wid], pare_v)
        pltpu.sync_copy(lvl_hbm.at[wid], lvl_v)

        def fire(c, slot):
            ti = table_hbm.bitcast(jnp.int32)
            return [
                pltpu.async_copy(
                    ti.at[idx2_v.at[c, 0]], bufs.at[slot, 0], sems[slot]),
                pltpu.async_copy(
                    ti.at[idx2_v.at[c, 1]], bufs.at[slot, 1], sems[slot]),
                pltpu.async_copy(
                    ti.at[idxe_v.at[c]], bufe.at[slot], sems[slot]),
            ]

        def compute_chunk(c, slot):
            bAB = bufs.at[slot, 0]
            bCD = bufs.at[slot, 1]
            bE = bufe.at[slot]
            himask = jnp.full((16,), -65536, jnp.int32)

            def group(g, _):
                rows = g * 16 + lax.iota(jnp.int32, 16)
                rows2 = rows + CHUNK
                zv = jnp.zeros((16,), jnp.int32)
                sh_a = (1 - par2_v[c, 0, pl.ds(g * 16, 16)]) * 16
                sh_b = (1 - par2_v[c, 0, pl.ds(CHUNK + g * 16, 16)]) * 16
                sh_c = (1 - par2_v[c, 1, pl.ds(g * 16, 16)]) * 16
                sh_d = (1 - par2_v[c, 1, pl.ds(CHUNK + g * 16, 16)]) * 16
                sh_e = (1 - pare_v[c, pl.ds(g * 16, 16)]) * 16

                def bf(w, sh):
                    return plsc.bitcast((w << sh) & himask, jnp.float32)

                lane = lax.iota(jnp.int32, 16)
                d63 = jnp.full((16,), 63, jnp.int32)

                def dim_body(i, accs):
                    sa, sb, sab, sc_, sd, scd = accs
                    for k in range(2):
                        dv = (lane + (i * 2 + k)) & d63
                        a = bf(plsc.load_gather(bAB, [rows, zv, dv]), sh_a)
                        b = bf(plsc.load_gather(bAB, [rows2, zv, dv]), sh_b)
                        cc = bf(plsc.load_gather(bCD, [rows, zv, dv]), sh_c)
                        dd = bf(plsc.load_gather(bCD, [rows2, zv, dv]), sh_d)
                        eab = a - b
                        ecd = cc - dd
                        sa = sa + a * a
                        sb = sb + b * b
                        sab = sab + eab * eab
                        sc_ = sc_ + cc * cc
                        sd = sd + dd * dd
                        scd = scd + ecd * ecd
                    return (sa, sb, sab, sc_, sd, scd)

                z = jnp.zeros((16,), jnp.float32)
                sa, sb, sab, sc_, sd, scd = lax.fori_loop(
                    0, EMB // 2, dim_body, (z, z, z, z, z, z))

                c64 = jnp.full((16,), EMB, jnp.int32)
                ra = jnp.abs(bf(plsc.load_gather(bAB, [rows, zv, c64]), sh_a))
                rb = jnp.abs(bf(plsc.load_gather(bAB, [rows2, zv, c64]), sh_b))
                rc = jnp.abs(bf(plsc.load_gather(bCD, [rows, zv, c64]), sh_c))
                rd = jnp.abs(bf(plsc.load_gather(bCD, [rows2, zv, c64]), sh_d))
                re = jnp.abs(bf(plsc.load_gather(bE, [rows, zv, c64]), sh_e))

                euc = _sqrt(sab)
                na = _sqrt(sa)
                nb = _sqrt(sb)
                loss1 = (jnp.maximum(euc + ra - rb - MARGIN, 0.0)
                         + jnp.abs(na - 1.0) + jnp.abs(nb - 1.0))

                dst = _sqrt(scd)
                nc_ = _sqrt(sc_)
                nd = _sqrt(sd)
                lossd = (jnp.maximum(rc + rd - dst + MARGIN, 0.0)
                         + jnp.abs(nc_ - 1.0) + jnp.abs(nd - 1.0))

                off = c * CHUNK + g * 16
                lvl16 = lvl_v[pl.ds(off, 16)].astype(jnp.float32)
                lossr = jnp.maximum(_sqrt(lvl16) * 0.5 - re, 0.0)

                out_v[pl.ds(off, 16)] = loss1 + lossd + lossr
                return 0

            lax.fori_loop(0, CHUNK // 16, group, 0)

        pending = {s: [] for s in range(NBUF)}
        for s in range(NBUF - 1):
            pending[s] = fire(s, s)
        for c in range(NCHUNK):
            slot = c % NBUF
            for cp in pending[slot]:
                cp.wait()
            nxt = c + NBUF - 1
            if nxt < NCHUNK:
                pending[nxt % NBUF] = fire(nxt, nxt % NBUF)
            compute_chunk(c, slot)

        pltpu.sync_copy(out_v, out_hbm.at[pl.ds(wid * PER_TILE, PER_TILE)])

    return k(table, idx2, par2, idxe, pare, lvl)


def kernel(nf1, nf1_neg, dis, cl, cls_table):
    del nf1_neg
    ab = jnp.concatenate(
        [nf1[:, 0].reshape(NW, NCHUNK, CHUNK),
         nf1[:, 1].reshape(NW, NCHUNK, CHUNK)], axis=-1)
    cd = jnp.concatenate(
        [dis[:, 0].reshape(NW, NCHUNK, CHUNK),
         dis[:, 1].reshape(NW, NCHUNK, CHUNK)], axis=-1)
    full = jnp.stack([ab, cd], axis=2)
    idx2 = full >> 1
    par2 = full & 1
    e = cl[:, 0].reshape(NW, NCHUNK, CHUNK)
    idxe = e >> 1
    pare = e & 1
    lvl = cl[:, 1].reshape(NW, PER_TILE)
    table_bf = _repack(cls_table.T)
    table3 = table_bf.reshape(1000000 // 2, 2, ROW)
    out = _sc_loss(table3, idx2, par2, idxe, pare, lvl)
    return out.reshape(BATCH, 1)
